# Initial kernel scaffold; baseline (speedup 1.0000x reference)
import jax
import jax.numpy as jnp
from jax.experimental import pallas as pl


def kernel(x, index, v, scale):
    raise NotImplementedError("write your pallas kernel here")


# speedup vs baseline: 41.5228x; 41.5228x over previous
import functools

import jax
import jax.numpy as jnp
import numpy as np
from jax.experimental import pallas as pl
from jax.experimental.pallas import tpu as pltpu

N = 50000
NUM_FEATURES = 3
NUM_THETAS = 32
BUMP_STEPS = 32
R = 1.1
NUM_SEGMENTS = 128
BT = BUMP_STEPS * NUM_THETAS

C = 1024


def _body(x_ref, idx_ref, vt_ref, lin_ref, out_ref):
    i = pl.program_id(0)
    nh = jnp.dot(x_ref[...], vt_ref[...], preferred_element_type=jnp.float32)
    ecc = jax.nn.sigmoid(lin_ref[...] - nh)
    ids = idx_ref[0, 0, :]
    rows = jax.lax.broadcasted_iota(jnp.int32, (NUM_SEGMENTS, C), 0)
    onehot = (rows == ids[None, :]).astype(jnp.float32)
    part = jnp.dot(onehot, ecc, preferred_element_type=jnp.float32)

    @pl.when(i == 0)
    def _init():
        out_ref[...] = part

    @pl.when(i > 0)
    def _acc():
        out_ref[...] += part


@jax.jit
def kernel(x, index, v, scale):
    n = x.shape[0]
    npad = ((n + C - 1) // C) * C
    g = npad // C
    scale_f = jnp.asarray(scale, jnp.float32)
    lin = np.linspace(-R, R, BUMP_STEPS, dtype=np.float32)
    lin2 = jnp.asarray(np.repeat(lin, NUM_THETAS).reshape(1, BT)) * scale_f
    vt = jnp.tile(v * scale_f, (1, BUMP_STEPS))
    xp = jnp.pad(x, ((0, npad - n), (0, 0)))
    idxp = jnp.pad(index, (0, npad - n), constant_values=NUM_SEGMENTS)
    idx3 = idxp.reshape(g, 1, C)

    out = pl.pallas_call(
        _body,
        grid=(g,),
        in_specs=[
            pl.BlockSpec((C, NUM_FEATURES), lambda i: (i, 0)),
            pl.BlockSpec((1, 1, C), lambda i: (i, 0, 0)),
            pl.BlockSpec((NUM_FEATURES, BT), lambda i: (0, 0)),
            pl.BlockSpec((1, BT), lambda i: (0, 0)),
        ],
        out_specs=pl.BlockSpec((NUM_SEGMENTS, BT), lambda i: (0, 0)),
        out_shape=jax.ShapeDtypeStruct((NUM_SEGMENTS, BT), jnp.float32),
        compiler_params=pltpu.CompilerParams(
            dimension_semantics=("arbitrary",),
        ),
    )(xp, idx3, vt, lin2)
    return out.reshape(NUM_SEGMENTS, BUMP_STEPS, NUM_THETAS)


# speedup vs baseline: 61.2682x; 1.4755x over previous
import functools

import jax
import jax.numpy as jnp
import numpy as np
from jax.experimental import pallas as pl
from jax.experimental.pallas import tpu as pltpu

N = 50000
NUM_FEATURES = 3
NUM_THETAS = 32
BUMP_STEPS = 32
R = 1.1
NUM_SEGMENTS = 128
BT = BUMP_STEPS * NUM_THETAS

C = 1024


def _body(x_ref, idx_ref, vt_ref, lin_ref, out_ref):
    i = pl.program_id(0)
    nh = jnp.dot(x_ref[...], vt_ref[...], preferred_element_type=jnp.float32)
    u = lin_ref[...] - nh
    ecc = (1.0 + jnp.tanh(u)).astype(jnp.bfloat16)
    ids = idx_ref[0, 0, :]
    rows = jax.lax.broadcasted_iota(jnp.int32, (NUM_SEGMENTS, C), 0)
    onehot = (rows == ids[None, :]).astype(jnp.bfloat16) * jnp.bfloat16(0.5)
    part = jnp.dot(onehot, ecc, preferred_element_type=jnp.float32)

    @pl.when(i == 0)
    def _init():
        out_ref[...] = part

    @pl.when(i > 0)
    def _acc():
        out_ref[...] += part


@jax.jit
def kernel(x, index, v, scale):
    n = x.shape[0]
    npad = ((n + C - 1) // C) * C
    g = npad // C
    half_scale = jnp.asarray(scale, jnp.float32) * 0.5
    lin = np.linspace(-R, R, BUMP_STEPS, dtype=np.float32)
    lin2 = jnp.asarray(np.repeat(lin, NUM_THETAS).reshape(1, BT)) * half_scale
    vt = jnp.tile(v * half_scale, (1, BUMP_STEPS)).astype(jnp.bfloat16)
    xp = jnp.pad(x, ((0, npad - n), (0, 0))).astype(jnp.bfloat16)
    idxp = jnp.pad(index, (0, npad - n), constant_values=NUM_SEGMENTS)
    idx3 = idxp.reshape(g, 1, C)

    out = pl.pallas_call(
        _body,
        grid=(g,),
        in_specs=[
            pl.BlockSpec((C, NUM_FEATURES), lambda i: (i, 0)),
            pl.BlockSpec((1, 1, C), lambda i: (i, 0, 0)),
            pl.BlockSpec((NUM_FEATURES, BT), lambda i: (0, 0)),
            pl.BlockSpec((1, BT), lambda i: (0, 0)),
        ],
        out_specs=pl.BlockSpec((NUM_SEGMENTS, BT), lambda i: (0, 0)),
        out_shape=jax.ShapeDtypeStruct((NUM_SEGMENTS, BT), jnp.float32),
        compiler_params=pltpu.CompilerParams(
            dimension_semantics=("arbitrary",),
        ),
    )(xp, idx3, vt, lin2)
    return out.reshape(NUM_SEGMENTS, BUMP_STEPS, NUM_THETAS)


# speedup vs baseline: 66.5628x; 1.0864x over previous
import functools

import jax
import jax.numpy as jnp
import numpy as np
from jax.experimental import pallas as pl
from jax.experimental.pallas import tpu as pltpu

N = 50000
NUM_FEATURES = 3
NUM_THETAS = 32
BUMP_STEPS = 32
R = 1.1
NUM_SEGMENTS = 128
BT = BUMP_STEPS * NUM_THETAS

C = 2048


def _body(x_ref, idx_ref, vt_ref, lin_ref, out_ref):
    i = pl.program_id(0)
    nh = jnp.dot(x_ref[...], vt_ref[...], preferred_element_type=jnp.float32)
    u = lin_ref[...] - nh
    ecc = (1.0 + jnp.tanh(u)).astype(jnp.bfloat16)
    ids = idx_ref[0, 0, :]
    rows = jax.lax.broadcasted_iota(jnp.int32, (NUM_SEGMENTS, C), 0)
    onehot = (rows == ids[None, :]).astype(jnp.bfloat16) * jnp.bfloat16(0.5)
    part = jnp.dot(onehot, ecc, preferred_element_type=jnp.float32)

    @pl.when(i == 0)
    def _init():
        out_ref[...] = part

    @pl.when(i > 0)
    def _acc():
        out_ref[...] += part


@jax.jit
def kernel(x, index, v, scale):
    n = x.shape[0]
    npad = ((n + C - 1) // C) * C
    g = npad // C
    half_scale = jnp.asarray(scale, jnp.float32) * 0.5
    lin = np.linspace(-R, R, BUMP_STEPS, dtype=np.float32)
    lin2 = jnp.asarray(np.repeat(lin, NUM_THETAS).reshape(1, BT)) * half_scale
    vt = jnp.tile(v * half_scale, (1, BUMP_STEPS)).astype(jnp.bfloat16)
    xp = jnp.pad(x, ((0, npad - n), (0, 0))).astype(jnp.bfloat16)
    idxp = jnp.pad(index, (0, npad - n), constant_values=NUM_SEGMENTS)
    idx3 = idxp.reshape(g, 1, C)

    out = pl.pallas_call(
        _body,
        grid=(g,),
        in_specs=[
            pl.BlockSpec((C, NUM_FEATURES), lambda i: (i, 0)),
            pl.BlockSpec((1, 1, C), lambda i: (i, 0, 0)),
            pl.BlockSpec((NUM_FEATURES, BT), lambda i: (0, 0)),
            pl.BlockSpec((1, BT), lambda i: (0, 0)),
        ],
        out_specs=pl.BlockSpec((NUM_SEGMENTS, BT), lambda i: (0, 0)),
        out_shape=jax.ShapeDtypeStruct((NUM_SEGMENTS, BT), jnp.float32),
        compiler_params=pltpu.CompilerParams(
            dimension_semantics=("arbitrary",),
        ),
    )(xp, idx3, vt, lin2)
    return out.reshape(NUM_SEGMENTS, BUMP_STEPS, NUM_THETAS)
